# Initial kernel scaffold; baseline (speedup 1.0000x reference)
import jax
import jax.numpy as jnp
from jax.experimental import pallas as pl


def kernel(x, edge_index, edge_type, rel_emb, W, b):
    raise NotImplementedError("write your pallas kernel here")


# speedup vs baseline: 3.1818x; 3.1818x over previous
import functools

import jax
import jax.numpy as jnp
from jax import lax
from jax.experimental import pallas as pl
from jax.experimental.pallas import tpu as pltpu
from jax.experimental.pallas import tpu_sc as plsc

N_NODES = 10000
D = 128
DH = 64
NC = 2
NS = 16
NW = NC * NS
CH = 128
AGG_ROWS = 10240
ZCOPIES = 5
OPT8 = (N_NODES // NS) // 8 * 8


def _sc_agg_call(kch):
    mesh = plsc.VectorSubcoreMesh(core_axis_name="c", subcore_axis_name="s")

    @functools.partial(
        pl.kernel,
        mesh=mesh,
        compiler_params=pltpu.CompilerParams(use_tc_tiling_on_sc=False),
        out_type=jax.ShapeDtypeStruct((NC, 2, N_NODES, DH), jnp.float32),
        scratch_types=[
            pltpu.VMEM((kch, CH), jnp.int32),
            pltpu.VMEM((kch, CH), jnp.int32),
            pltpu.VMEM((kch, CH), jnp.int32),
            pltpu.VMEM((CH, DH), jnp.float32),
            pltpu.VMEM((CH, DH), jnp.float32),
            pltpu.VMEM_SHARED((AGG_ROWS, DH), jnp.float32),
            pltpu.SemaphoreType.DMA,
            pltpu.SemaphoreType.DMA,
        ],
    )
    def sc_agg(src_hbm, typ_hbm, dst_hbm, x0_hbm, x1_hbm, r0_hbm, r1_hbm,
               part_hbm, src_v, typ_v, dst_v, xb, rb, agg_sh, sem1, sem2):
        c = lax.axis_index("c")
        s = lax.axis_index("s")
        wid = c * NS + s

        pltpu.sync_copy(src_hbm.at[wid], src_v)
        pltpu.sync_copy(typ_hbm.at[wid], typ_v)
        pltpu.sync_copy(dst_hbm.at[wid], dst_v)

        zero = jnp.zeros((16,), jnp.float32)

        for p, (xh, rh) in enumerate(((x0_hbm, r0_hbm), (x1_hbm, r1_hbm))):
            def zrow(r, carry):
                for k in range(DH // 16):
                    xb[r, pl.ds(k * 16, 16)] = zero
                return carry

            lax.fori_loop(0, CH, zrow, 0)
            for j in range(ZCOPIES):
                pltpu.sync_copy(
                    xb, agg_sh.at[pl.ds(s * (ZCOPIES * CH) + j * CH, CH)])
            plsc.subcore_barrier()

            def chunk(ch, carry):
                cp1 = pltpu.async_copy(xh.at[src_v.at[ch]], xb, sem1)
                cp2 = pltpu.async_copy(rh.at[typ_v.at[ch]], rb, sem2)
                cp1.wait()
                cp2.wait()

                def mul(r, inner):
                    for k in range(DH // 16):
                        sl = pl.ds(k * 16, 16)
                        xb[r, sl] = xb[r, sl] * rb[r, sl]
                    return inner

                lax.fori_loop(0, CH, mul, 0)
                pltpu.sync_copy(xb, agg_sh.at[dst_v.at[ch]], add=True)
                return carry

            lax.fori_loop(0, kch, chunk, 0)
            plsc.subcore_barrier()

            pltpu.sync_copy(agg_sh.at[pl.ds(s * OPT8, OPT8)],
                            part_hbm.at[c].at[p].at[pl.ds(s * OPT8, OPT8)])

            @pl.when(s == 0)
            def _():
                base = NS * OPT8
                pltpu.sync_copy(agg_sh.at[pl.ds(base, N_NODES - base)],
                                part_hbm.at[c].at[p].at[pl.ds(base, N_NODES - base)])

            plsc.subcore_barrier()

    return sc_agg


def _tc_body(p_ref, x_ref, w_ref, b_ref, o_ref):
    agg = jnp.concatenate(
        [p_ref[0, 0] + p_ref[1, 0], p_ref[0, 1] + p_ref[1, 1]], axis=1)
    y = jnp.dot(agg, w_ref[...], preferred_element_type=jnp.float32,
                precision=lax.Precision.HIGHEST)
    o_ref[...] = jnp.maximum(y + b_ref[...] + x_ref[...], 0.0)


def _tc_update(parts, x, w, b2d):
    blk = 1000
    grid = (N_NODES // blk,)
    return pl.pallas_call(
        _tc_body,
        grid=grid,
        in_specs=[
            pl.BlockSpec((NC, 2, blk, DH), lambda i: (0, 0, i, 0)),
            pl.BlockSpec((blk, D), lambda i: (i, 0)),
            pl.BlockSpec((D, D), lambda i: (0, 0)),
            pl.BlockSpec((1, D), lambda i: (0, 0)),
        ],
        out_specs=pl.BlockSpec((blk, D), lambda i: (i, 0)),
        out_shape=jax.ShapeDtypeStruct((N_NODES, D), jnp.float32),
    )(parts, x, w, b2d)


def kernel(x, edge_index, edge_type, rel_emb, W, b):
    n_edges = edge_index.shape[1]
    kch = -(-n_edges // (NW * CH))
    e_pad = NW * kch * CH
    pad = e_pad - n_edges

    src = edge_index[0].astype(jnp.int32)
    dst = edge_index[1].astype(jnp.int32)
    typ = edge_type.astype(jnp.int32)
    if pad:
        src = jnp.concatenate([src, jnp.zeros((pad,), jnp.int32)])
        typ = jnp.concatenate([typ, jnp.zeros((pad,), jnp.int32)])
        dst = jnp.concatenate([dst, jnp.full((pad,), N_NODES, jnp.int32)])
    src = src.reshape(NW, kch, CH)
    typ = typ.reshape(NW, kch, CH)
    dst = dst.reshape(NW, kch, CH)

    x0 = x[:, :DH]
    x1 = x[:, DH:]
    r0 = rel_emb[:, :DH]
    r1 = rel_emb[:, DH:]

    parts = _sc_agg_call(kch)(src, typ, dst, x0, x1, r0, r1)
    return _tc_update(parts, x, W, b.reshape(1, D))
